# baseline (speedup 1.0000x reference)
import jax
import jax.numpy as jnp
from jax import lax
from jax.experimental import pallas as pl

NUM_CLASSES = 1000
ROWS_PER_BLOCK = 1024


def _onehot_block(tgt_ref, out_ref):
    tgt = tgt_ref[0, 0, :]
    iota = lax.broadcasted_iota(jnp.int32, (ROWS_PER_BLOCK, NUM_CLASSES), 1)
    out_ref[:, :] = (iota == tgt[:, None]).astype(jnp.float32)


def kernel(target):
    b, s = target.shape
    n = b * s
    num_blocks = n // ROWS_PER_BLOCK
    flat = target.reshape(num_blocks, 1, ROWS_PER_BLOCK)
    out = pl.pallas_call(
        _onehot_block,
        grid=(num_blocks,),
        in_specs=[pl.BlockSpec((1, 1, ROWS_PER_BLOCK), lambda i: (i, 0, 0))],
        out_specs=pl.BlockSpec((ROWS_PER_BLOCK, NUM_CLASSES), lambda i: (i, 0)),
        out_shape=jax.ShapeDtypeStruct((n, NUM_CLASSES), jnp.float32),
    )(flat)
    return out.reshape(b, s, NUM_CLASSES)


# speedup vs baseline: 1.3823x; 1.3823x over previous
import jax
import jax.numpy as jnp
from jax import lax
from jax.experimental import pallas as pl

NUM_CLASSES = 1000
BATCH_BLOCK = 128


def _onehot_block(tgt_ref, out_ref):
    tgt = tgt_ref[:, :]
    iota = lax.broadcasted_iota(
        jnp.int32, (tgt.shape[0], tgt.shape[1], NUM_CLASSES), 2)
    out_ref[:, :, :] = (iota == tgt[:, :, None]).astype(jnp.float32)


def kernel(target):
    b, s = target.shape
    num_blocks = b // BATCH_BLOCK
    return pl.pallas_call(
        _onehot_block,
        grid=(num_blocks,),
        in_specs=[pl.BlockSpec((BATCH_BLOCK, s), lambda i: (i, 0))],
        out_specs=pl.BlockSpec((BATCH_BLOCK, s, NUM_CLASSES), lambda i: (i, 0, 0)),
        out_shape=jax.ShapeDtypeStruct((b, s, NUM_CLASSES), jnp.float32),
    )(target)
